# baseline (speedup 1.0000x reference)
import functools

import jax
import jax.numpy as jnp
from jax import lax
from jax.experimental import pallas as pl
from jax.experimental.pallas import tpu as pltpu
from jax.experimental.pallas import tpu_sc as plsc

EMBED = 32
PATH = 20
LANES = 16


def _newton_rsqrt(x):
    i = plsc.bitcast(x, jnp.int32)
    y = plsc.bitcast(jnp.int32(0x5F3759DF) - lax.shift_right_logical(i, 1),
                     jnp.float32)
    for _ in range(3):
        y = y * (1.5 - 0.5 * x * y * y)
    return y


def _tree_prob_body(ce_hbm, idx_hbm, table_hbm, out_hbm,
                    ce_v, idx_v, rows_v, out_v, sem):
    cid = lax.axis_index("c")
    sid = lax.axis_index("s")

    @pl.when(jnp.logical_and(cid == 0, sid == 0))
    def _():
        pltpu.sync_copy(idx_hbm, idx_v)
        pltpu.sync_copy(ce_hbm, ce_v)
        pltpu.async_copy(table_hbm.at[idx_v], rows_v, sem).wait()

        lanes = lax.broadcasted_iota(jnp.int32, (LANES,), 0)
        ce_lo = ce_v[pl.ds(0, LANES)]
        ce_hi = ce_v[pl.ds(LANES, LANES)]
        cnorm2 = jnp.sum(ce_lo * ce_lo + ce_hi * ce_hi)

        d_vec0 = jnp.zeros((LANES,), jnp.float32)
        s_vec0 = jnp.ones((LANES,), jnp.float32)
        d_vec1 = jnp.zeros((LANES,), jnp.float32)
        s_vec1 = jnp.ones((LANES,), jnp.float32)
        for i in range(PATH):
            v1 = rows_v[i, pl.ds(0, LANES)]
            v2 = rows_v[i, pl.ds(LANES, LANES)]
            di = jnp.sum(v1 * ce_lo + v2 * ce_hi)
            si = jnp.sum(v1 * v1 + v2 * v2)
            lane_hit = lanes == (i % LANES)
            if i < LANES:
                d_vec0 = jnp.where(lane_hit, di, d_vec0)
                s_vec0 = jnp.where(lane_hit, si, s_vec0)
            else:
                d_vec1 = jnp.where(lane_hit, di, d_vec1)
                s_vec1 = jnp.where(lane_hit, si, s_vec1)

        p0 = (1.0 + d_vec0 * _newton_rsqrt(s_vec0 * cnorm2)) * 0.5
        p1 = (1.0 + d_vec1 * _newton_rsqrt(s_vec1 * cnorm2)) * 0.5
        p1 = jnp.where(lanes < PATH - LANES, p1, 1.0)
        p = p0 * p1

        for shift in (8, 4, 2, 1):
            shifted = p.at[jnp.bitwise_and(lanes + shift, LANES - 1)].get(
                mode="promise_in_bounds")
            p = p * shifted
        out_v[...] = p
        pltpu.sync_copy(out_v, out_hbm)


def kernel(context_embedding, input_path_idxs, table):
    ce = context_embedding.reshape(EMBED).astype(jnp.float32)
    idx = input_path_idxs.astype(jnp.int32)
    mesh = plsc.VectorSubcoreMesh(core_axis_name="c", subcore_axis_name="s")
    call = functools.partial(
        pl.kernel,
        out_type=jax.ShapeDtypeStruct((LANES,), jnp.float32),
        mesh=mesh,
        scratch_types=[
            pltpu.VMEM((EMBED,), jnp.float32),
            pltpu.VMEM((PATH,), jnp.int32),
            pltpu.VMEM((PATH, EMBED), jnp.float32),
            pltpu.VMEM((LANES,), jnp.float32),
            pltpu.SemaphoreType.DMA,
        ],
        compiler_params=pltpu.CompilerParams(
            use_tc_tiling_on_sc=False, needs_layout_passes=False),
    )(_tree_prob_body)
    out = call(ce, idx, table)
    return out[0]


# speedup vs baseline: 1.6938x; 1.6938x over previous
import functools

import jax
import jax.numpy as jnp
from jax import lax
from jax.experimental import pallas as pl
from jax.experimental.pallas import tpu as pltpu
from jax.experimental.pallas import tpu_sc as plsc

EMBED = 32
PATH = 20
LANES = 16


def _newton_rsqrt(x):
    i = plsc.bitcast(x, jnp.int32)
    y = plsc.bitcast(jnp.int32(0x5F3759DF) - lax.shift_right_logical(i, 1),
                     jnp.float32)
    for _ in range(3):
        y = y * (1.5 - 0.5 * x * y * y)
    return y


def _tree_prob_body(ce_hbm, idx_hbm, table_hbm, out_hbm,
                    ce_v, idx_v, rows_v, out_v, sem):
    cid = lax.axis_index("c")
    sid = lax.axis_index("s")

    @pl.when(jnp.logical_and(cid == 0, sid == 0))
    def _():
        pltpu.sync_copy(idx_hbm, idx_v)
        pltpu.sync_copy(ce_hbm, ce_v)
        i_lo = idx_v[pl.ds(0, LANES)]
        i_hi = idx_v[pl.ds(PATH - LANES, LANES)]
        copies = []
        for j in range(PATH):
            s = i_lo[j] if j < LANES else i_hi[j - (PATH - LANES)]
            copies.append(pltpu.async_copy(
                table_hbm.at[pl.ds(s, 1), :], rows_v.at[pl.ds(j, 1), :], sem))
        for c in copies:
            c.wait()

        lanes = lax.broadcasted_iota(jnp.int32, (LANES,), 0)
        ce_lo = ce_v[pl.ds(0, LANES)]
        ce_hi = ce_v[pl.ds(LANES, LANES)]
        cnorm2 = jnp.sum(ce_lo * ce_lo + ce_hi * ce_hi)

        d_vec0 = jnp.zeros((LANES,), jnp.float32)
        s_vec0 = jnp.ones((LANES,), jnp.float32)
        d_vec1 = jnp.zeros((LANES,), jnp.float32)
        s_vec1 = jnp.ones((LANES,), jnp.float32)
        for i in range(PATH):
            v1 = rows_v[i, pl.ds(0, LANES)]
            v2 = rows_v[i, pl.ds(LANES, LANES)]
            di = jnp.sum(v1 * ce_lo + v2 * ce_hi)
            si = jnp.sum(v1 * v1 + v2 * v2)
            lane_hit = lanes == (i % LANES)
            if i < LANES:
                d_vec0 = jnp.where(lane_hit, di, d_vec0)
                s_vec0 = jnp.where(lane_hit, si, s_vec0)
            else:
                d_vec1 = jnp.where(lane_hit, di, d_vec1)
                s_vec1 = jnp.where(lane_hit, si, s_vec1)

        p0 = (1.0 + d_vec0 * _newton_rsqrt(s_vec0 * cnorm2)) * 0.5
        p1 = (1.0 + d_vec1 * _newton_rsqrt(s_vec1 * cnorm2)) * 0.5
        p1 = jnp.where(lanes < PATH - LANES, p1, 1.0)
        p = p0 * p1

        for shift in (8, 4, 2, 1):
            shifted = p.at[jnp.bitwise_and(lanes + shift, LANES - 1)].get(
                mode="promise_in_bounds")
            p = p * shifted
        out_v[...] = p
        pltpu.sync_copy(out_v, out_hbm)


def kernel(context_embedding, input_path_idxs, table):
    ce = context_embedding.reshape(EMBED).astype(jnp.float32)
    idx = input_path_idxs.astype(jnp.int32)
    mesh = plsc.VectorSubcoreMesh(core_axis_name="c", subcore_axis_name="s")
    call = functools.partial(
        pl.kernel,
        out_type=jax.ShapeDtypeStruct((LANES,), jnp.float32),
        mesh=mesh,
        scratch_types=[
            pltpu.VMEM((EMBED,), jnp.float32),
            pltpu.VMEM((PATH,), jnp.int32),
            pltpu.VMEM((PATH, EMBED), jnp.float32),
            pltpu.VMEM((LANES,), jnp.float32),
            pltpu.SemaphoreType.DMA,
        ],
        compiler_params=pltpu.CompilerParams(
            use_tc_tiling_on_sc=True, needs_layout_passes=False),
    )(_tree_prob_body)
    out = call(ce, idx, table)
    return out[0]


# speedup vs baseline: 20.6516x; 12.1926x over previous
import functools

import jax
import jax.numpy as jnp
from jax import lax
from jax.experimental import pallas as pl
from jax.experimental.pallas import tpu as pltpu
from jax.experimental.pallas import tpu_sc as plsc

EMBED = 32
PATH = 20
LANES = 16


def _newton_rsqrt(x):
    i = plsc.bitcast(x, jnp.int32)
    y = plsc.bitcast(jnp.int32(0x5F3759DF) - lax.shift_right_logical(i, 1),
                     jnp.float32)
    for _ in range(3):
        y = y * (1.5 - 0.5 * x * y * y)
    return y


def _tree_prob_body(ce_hbm, idx_hbm, table_hbm, out_hbm,
                    ce_v, idx_v, blk_v, out_v, sem):
    cid = lax.axis_index("c")
    sid = lax.axis_index("s")

    @pl.when(jnp.logical_and(cid == 0, sid == 0))
    def _():
        pltpu.sync_copy(idx_hbm, idx_v.at[pl.ds(0, PATH)])
        pltpu.sync_copy(ce_hbm, ce_v)

        i_lo = idx_v[pl.ds(0, LANES)]
        i_hi = idx_v[pl.ds(LANES, LANES)]

        copies = []
        for j in range(PATH):
            s = i_lo[j] if j < LANES else i_hi[j - LANES]
            a = pl.multiple_of((s >> 7) << 7, 128)
            copies.append(pltpu.async_copy(
                table_hbm.at[:, pl.ds(a, 128)], blk_v.at[j], sem))
        for c in copies:
            c.wait()

        lanes = lax.broadcasted_iota(jnp.int32, (LANES,), 0)
        rows0 = lanes
        rows1 = jnp.minimum(lanes + LANES, PATH - 1)
        c0 = jnp.bitwise_and(i_lo, 127)
        c1 = jnp.bitwise_and(i_hi, 127)

        ce_lo = ce_v[pl.ds(0, LANES)]
        ce_hi = ce_v[pl.ds(LANES, LANES)]

        zero = jnp.zeros((LANES,), jnp.float32)
        acc_d0, acc_s0, acc_d1, acc_s1 = zero, zero, zero, zero
        for d in range(EMBED):
            cd = ce_lo[d] if d < LANES else ce_hi[d - LANES]
            dv = jnp.full((LANES,), d, jnp.int32)
            col0 = plsc.load_gather(blk_v, [rows0, dv, c0])
            col1 = plsc.load_gather(blk_v, [rows1, dv, c1])
            acc_d0 = acc_d0 + col0 * cd
            acc_s0 = acc_s0 + col0 * col0
            acc_d1 = acc_d1 + col1 * cd
            acc_s1 = acc_s1 + col1 * col1

        cnorm2 = jnp.sum(ce_lo * ce_lo + ce_hi * ce_hi)

        p0 = (1.0 + acc_d0 * _newton_rsqrt(acc_s0 * cnorm2)) * 0.5
        p1 = (1.0 + acc_d1 * _newton_rsqrt(acc_s1 * cnorm2)) * 0.5
        p1 = jnp.where(lanes < PATH - LANES, p1, 1.0)
        p = p0 * p1

        for shift in (8, 4, 2, 1):
            shifted = p.at[jnp.bitwise_and(lanes + shift, LANES - 1)].get(
                mode="promise_in_bounds")
            p = p * shifted
        out_v[...] = p
        pltpu.sync_copy(out_v, out_hbm)


def kernel(context_embedding, input_path_idxs, table):
    ce = context_embedding.reshape(EMBED).astype(jnp.float32)
    idx = input_path_idxs.astype(jnp.int32)
    mesh = plsc.VectorSubcoreMesh(core_axis_name="c", subcore_axis_name="s")
    call = functools.partial(
        pl.kernel,
        out_type=jax.ShapeDtypeStruct((LANES,), jnp.float32),
        mesh=mesh,
        scratch_types=[
            pltpu.VMEM((EMBED,), jnp.float32),
            pltpu.VMEM((2 * LANES,), jnp.int32),
            pltpu.VMEM((PATH, EMBED, 128), jnp.float32),
            pltpu.VMEM((LANES,), jnp.float32),
            pltpu.SemaphoreType.DMA,
        ],
        compiler_params=pltpu.CompilerParams(
            use_tc_tiling_on_sc=True, needs_layout_passes=False),
    )(_tree_prob_body)
    out = call(ce, idx, table.T)
    return out[0]


# speedup vs baseline: 21.0374x; 1.0187x over previous
import functools

import jax
import jax.numpy as jnp
from jax import lax
from jax.experimental import pallas as pl
from jax.experimental.pallas import tpu as pltpu
from jax.experimental.pallas import tpu_sc as plsc

EMBED = 32
PATH = 20
LANES = 16


def _newton_rsqrt(x):
    i = plsc.bitcast(x, jnp.int32)
    y = plsc.bitcast(jnp.int32(0x5F3759DF) - lax.shift_right_logical(i, 1),
                     jnp.float32)
    for _ in range(3):
        y = y * (1.5 - 0.5 * x * y * y)
    return y


def _tree_prob_body(ce_hbm, idx_hbm, table_hbm, out_hbm,
                    ce_v, idx_v, blk_v, out_v, sem):
    cid = lax.axis_index("c")
    sid = lax.axis_index("s")

    @pl.when(jnp.logical_and(cid == 0, sid == 0))
    def _():
        pltpu.sync_copy(idx_hbm, idx_v.at[pl.ds(0, PATH)])
        pltpu.sync_copy(ce_hbm, ce_v)

        i_lo = idx_v[pl.ds(0, LANES)]
        i_hi = idx_v[pl.ds(LANES, LANES)]

        copies = []
        for j in range(PATH):
            s = i_lo[j] if j < LANES else i_hi[j - LANES]
            a = pl.multiple_of((s >> 7) << 7, 128)
            copies.append(pltpu.async_copy(
                table_hbm.at[:, pl.ds(a, 128)], blk_v.at[j], sem))
        for c in copies:
            c.wait()

        lanes = lax.broadcasted_iota(jnp.int32, (LANES,), 0)
        rows0 = lanes
        rows1 = jnp.minimum(lanes + LANES, PATH - 1)
        c0 = jnp.bitwise_and(i_lo, 127)
        c1 = jnp.bitwise_and(i_hi, 127)

        ce_lo = ce_v[pl.ds(0, LANES)]
        ce_hi = ce_v[pl.ds(LANES, LANES)]

        def step(d, carry):
            a_d0, a_s0, a_d1, a_s1 = carry
            dv = jnp.full((LANES,), d, jnp.int32)
            d15 = jnp.bitwise_and(dv, LANES - 1)
            ced = jnp.where(
                d < LANES,
                ce_lo.at[d15].get(mode="promise_in_bounds"),
                ce_hi.at[d15].get(mode="promise_in_bounds"))
            col0 = plsc.load_gather(blk_v, [rows0, dv, c0])
            col1 = plsc.load_gather(blk_v, [rows1, dv, c1])
            return (a_d0 + col0 * ced, a_s0 + col0 * col0,
                    a_d1 + col1 * ced, a_s1 + col1 * col1)

        zero = jnp.zeros((LANES,), jnp.float32)
        acc_d0, acc_s0, acc_d1, acc_s1 = lax.fori_loop(
            0, EMBED, step, (zero, zero, zero, zero))

        cnorm2 = jnp.sum(ce_lo * ce_lo + ce_hi * ce_hi)

        p0 = (1.0 + acc_d0 * _newton_rsqrt(acc_s0 * cnorm2)) * 0.5
        p1 = (1.0 + acc_d1 * _newton_rsqrt(acc_s1 * cnorm2)) * 0.5
        p1 = jnp.where(lanes < PATH - LANES, p1, 1.0)
        p = p0 * p1

        for shift in (8, 4, 2, 1):
            shifted = p.at[jnp.bitwise_and(lanes + shift, LANES - 1)].get(
                mode="promise_in_bounds")
            p = p * shifted
        out_v[...] = p
        pltpu.sync_copy(out_v, out_hbm)


def kernel(context_embedding, input_path_idxs, table):
    ce = context_embedding.reshape(EMBED).astype(jnp.float32)
    idx = input_path_idxs.astype(jnp.int32)
    mesh = plsc.VectorSubcoreMesh(core_axis_name="c", subcore_axis_name="s")
    call = functools.partial(
        pl.kernel,
        out_type=jax.ShapeDtypeStruct((LANES,), jnp.float32),
        mesh=mesh,
        scratch_types=[
            pltpu.VMEM((EMBED,), jnp.float32),
            pltpu.VMEM((2 * LANES,), jnp.int32),
            pltpu.VMEM((PATH, EMBED, 128), jnp.float32),
            pltpu.VMEM((LANES,), jnp.float32),
            pltpu.SemaphoreType.DMA,
        ],
        compiler_params=pltpu.CompilerParams(
            use_tc_tiling_on_sc=True, needs_layout_passes=False),
    )(_tree_prob_body)
    out = call(ce, idx, table.T)
    return out[0]


# speedup vs baseline: 22.4484x; 1.0671x over previous
import functools

import jax
import jax.numpy as jnp
from jax import lax
from jax.experimental import pallas as pl
from jax.experimental.pallas import tpu as pltpu
from jax.experimental.pallas import tpu_sc as plsc

EMBED = 32
PATH = 20
LANES = 16


def _newton_rsqrt(x):
    i = plsc.bitcast(x, jnp.int32)
    y = plsc.bitcast(jnp.int32(0x5F3759DF) - lax.shift_right_logical(i, 1),
                     jnp.float32)
    for _ in range(3):
        y = y * (1.5 - 0.5 * x * y * y)
    return y


def _tree_prob_body(ce_hbm, idx_hbm, table_hbm, out_hbm,
                    ce_v, idx_v, blk_v, out_v, sem):
    cid = lax.axis_index("c")
    sid = lax.axis_index("s")

    @pl.when(jnp.logical_and(cid == 0, sid == 0))
    def _():
        pltpu.sync_copy(idx_hbm, idx_v.at[pl.ds(0, PATH)])
        pltpu.sync_copy(ce_hbm, ce_v)

        i_lo = idx_v[pl.ds(0, LANES)]
        i_hi = idx_v[pl.ds(LANES, LANES)]

        copies = []
        for j in range(PATH):
            s = i_lo[j] if j < LANES else i_hi[j - LANES]
            a = pl.multiple_of((s >> 7) << 7, 128)
            copies.append(pltpu.async_copy(
                table_hbm.at[:, pl.ds(a, 128)], blk_v.at[j], sem))
        for c in copies:
            c.wait()

        lanes = lax.broadcasted_iota(jnp.int32, (LANES,), 0)
        rows0 = lanes
        rows1 = jnp.minimum(lanes + LANES, PATH - 1)
        c0 = jnp.bitwise_and(i_lo, 127)
        c1 = jnp.bitwise_and(i_hi, 127)

        ce_lo = ce_v[pl.ds(0, LANES)]
        ce_hi = ce_v[pl.ds(LANES, LANES)]

        def step(d, carry):
            a_d0, a_s0, a_d1, a_s1 = carry
            dv = jnp.full((LANES,), d, jnp.int32)
            d15 = jnp.bitwise_and(dv, LANES - 1)
            ced = jnp.where(
                d < LANES,
                ce_lo.at[d15].get(mode="promise_in_bounds"),
                ce_hi.at[d15].get(mode="promise_in_bounds"))
            col0 = plsc.load_gather(blk_v, [rows0, dv, c0])
            col1 = plsc.load_gather(blk_v, [rows1, dv, c1])
            return (a_d0 + col0 * ced, a_s0 + col0 * col0,
                    a_d1 + col1 * ced, a_s1 + col1 * col1)

        zero = jnp.zeros((LANES,), jnp.float32)
        acc_d0, acc_s0, acc_d1, acc_s1 = lax.fori_loop(
            0, EMBED, step, (zero, zero, zero, zero))

        cnorm2 = jnp.sum(ce_lo * ce_lo + ce_hi * ce_hi)

        p0 = (1.0 + acc_d0 * _newton_rsqrt(acc_s0 * cnorm2)) * 0.5
        p1 = (1.0 + acc_d1 * _newton_rsqrt(acc_s1 * cnorm2)) * 0.5
        p1 = jnp.where(lanes < PATH - LANES, p1, 1.0)
        p = p0 * p1

        for shift in (8, 4, 2, 1):
            shifted = p.at[jnp.bitwise_and(lanes + shift, LANES - 1)].get(
                mode="promise_in_bounds")
            p = p * shifted
        out_v[...] = p
        pltpu.sync_copy(out_v, out_hbm)


def kernel(context_embedding, input_path_idxs, table):
    ce = context_embedding.reshape(EMBED).astype(jnp.float32)
    idx = input_path_idxs.astype(jnp.int32)
    mesh = plsc.VectorSubcoreMesh(core_axis_name="c", subcore_axis_name="s",
                                  num_cores=1)
    call = functools.partial(
        pl.kernel,
        out_type=jax.ShapeDtypeStruct((LANES,), jnp.float32),
        mesh=mesh,
        scratch_types=[
            pltpu.VMEM((EMBED,), jnp.float32),
            pltpu.VMEM((2 * LANES,), jnp.int32),
            pltpu.VMEM((PATH, EMBED, 128), jnp.float32),
            pltpu.VMEM((LANES,), jnp.float32),
            pltpu.SemaphoreType.DMA,
        ],
        compiler_params=pltpu.CompilerParams(
            use_tc_tiling_on_sc=True, needs_layout_passes=False),
    )(_tree_prob_body)
    out = call(ce, idx, table.T)
    return out[0]
